# baseline (speedup 1.0000x reference)
import jax
import jax.numpy as jnp
from jax import lax
from jax.experimental import pallas as pl
from jax.experimental.pallas import tpu as pltpu
from jax.experimental.pallas import tpu_sc as plsc

D = 32
B = 4096
HIST = 50
NC = 2
NS = 16
L = 16
NW = NC * NS
ROWS_W = B // NW
PAIRS_W = ROWS_W * HIST
CHUNK = 128
NCHUNK = PAIRS_W // CHUNK
GROUP = 4
GW = D * GROUP


def _iota16():
    return lax.broadcasted_iota(jnp.int32, (L,), 0)


def _sqrt16(w):
    bits = plsc.bitcast(w, jnp.int32)
    r = plsc.bitcast(jnp.int32(0x5F3759DF) - (bits >> 1), jnp.float32)
    hw = 0.5 * w
    r = r * (1.5 - hw * r * r)
    r = r * (1.5 - hw * r * r)
    r = r * (1.5 - hw * r * r)
    return w * r


def _neg_acosh1p(t):
    u = t + _sqrt16(t * (2.0 + t))
    poly = 1.0 - u * (0.5 - u * (1.0 / 3.0 - u * (0.25 - u * 0.2)))
    return -(u * poly)


def _tile_body(matrix, items_g, items_lo, origin_g, origin_lo, out,
               idx_v, lo_v, oidx_v, ylo_v, y_rows, x0, x1,
               ny_v, out_v, sem_y, sem0, sem1):
    wid = lax.axis_index("s") * NC + lax.axis_index("c")
    pltpu.sync_copy(items_g.at[wid], idx_v)
    pltpu.sync_copy(items_lo.at[wid], lo_v)
    pltpu.sync_copy(origin_g.at[wid], oidx_v)
    pltpu.sync_copy(origin_lo.at[wid], ylo_v)
    y_cp = pltpu.async_copy(matrix.at[oidx_v], y_rows, sem_y)
    pltpu.async_copy(matrix.at[idx_v.at[0]], x0, sem0)
    pltpu.async_copy(matrix.at[idx_v.at[1]], x1, sem1)
    iota = _iota16()
    y_cp.wait()

    for g8 in range(ROWS_W // L):
        rows = iota + (g8 * L)
        lo = plsc.load_gather(ylo_v, [rows])
        acc = jnp.zeros((L,), jnp.float32)
        for d in range(D):
            dd = (iota + d) & (D - 1)
            yd = plsc.load_gather(y_rows, [rows, lo + dd])
            acc = acc + yd * yd
        ny_v[pl.ds(g8 * L, L)] = acc

    def chunk_compute(j, x_buf):
        jv = iota * 0 + j
        for g in range(CHUNK // L):
            rows_x = iota + (g * L)
            p = j * CHUNK + (g * L) + iota
            b = (p * 5243) >> 18
            ny = plsc.load_gather(ny_v, [b])
            ylo = plsc.load_gather(ylo_v, [b])
            xlo = plsc.load_gather(lo_v, [jv, rows_x])
            sq = jnp.zeros((L,), jnp.float32)
            nx = jnp.zeros((L,), jnp.float32)
            for d in range(D):
                dd = (iota + d) & (D - 1)
                xd = plsc.load_gather(x_buf, [rows_x, xlo + dd])
                yd = plsc.load_gather(y_rows, [b, ylo + dd])
                df = xd - yd
                sq = sq + df * df
                nx = nx + xd * xd
            denom = jnp.maximum((1.0 - nx) * (1.0 - ny), 1e-7)
            arg = 1.0 + (2.0 * sq) / denom
            arg = jnp.maximum(arg, 1.0 + 1e-7)
            out_v[pl.ds(j * CHUNK + g * L, L)] = _neg_acosh1p(arg - 1.0)

    def pair_body(i, carry):
        for b, (xb, semb) in enumerate(((x0, sem0), (x1, sem1))):
            j = 2 * i + b
            pltpu.make_async_copy(matrix.at[idx_v.at[j]], xb, semb).wait()
            chunk_compute(j, xb)
            nj = j + 2

            @pl.when(nj < NCHUNK)
            def _():
                pltpu.async_copy(matrix.at[idx_v.at[nj]], xb, semb)
        return carry

    lax.fori_loop(0, NCHUNK // 2, pair_body, 0)
    pltpu.sync_copy(out_v, out.at[pl.ds(wid * PAIRS_W, PAIRS_W)])


def kernel(matrix, items, origin_item):
    n_rows = matrix.shape[0]
    matrix4 = matrix.reshape(n_rows // GROUP, GW)
    items_g = (items >> 2).reshape(NW, NCHUNK, CHUNK)
    items_lo = ((items & 3) * D).reshape(NW, NCHUNK, CHUNK)
    origin_g = (origin_item >> 2).reshape(NW, ROWS_W)
    origin_lo = ((origin_item & 3) * D).reshape(NW, ROWS_W)
    mesh = plsc.VectorSubcoreMesh(core_axis_name="c", subcore_axis_name="s")
    f = pl.kernel(
        _tile_body,
        out_type=jax.ShapeDtypeStruct((B * HIST,), jnp.float32),
        mesh=mesh,
        scratch_types=[
            pltpu.VMEM((NCHUNK, CHUNK), jnp.int32),
            pltpu.VMEM((NCHUNK, CHUNK), jnp.int32),
            pltpu.VMEM((ROWS_W,), jnp.int32),
            pltpu.VMEM((ROWS_W,), jnp.int32),
            pltpu.VMEM((ROWS_W, GW), jnp.float32),
            pltpu.VMEM((CHUNK, GW), jnp.float32),
            pltpu.VMEM((CHUNK, GW), jnp.float32),
            pltpu.VMEM((ROWS_W,), jnp.float32),
            pltpu.VMEM((PAIRS_W,), jnp.float32),
            pltpu.SemaphoreType.DMA,
            pltpu.SemaphoreType.DMA,
            pltpu.SemaphoreType.DMA,
        ],
        compiler_params=pltpu.CompilerParams(
            needs_layout_passes=False, use_tc_tiling_on_sc=True),
    )
    out = f(matrix4, items_g, items_lo, origin_g, origin_lo)
    return out.reshape(B, HIST)


# speedup vs baseline: 1.5312x; 1.5312x over previous
import jax
import jax.numpy as jnp
from jax import lax
from jax.experimental import pallas as pl
from jax.experimental.pallas import tpu as pltpu
from jax.experimental.pallas import tpu_sc as plsc

D = 32
B = 4096
HIST = 50
NC = 2
NS = 16
L = 16
NW = NC * NS
ROWS_W = B // NW
PAIRS_W = ROWS_W * HIST
CHUNK = 128
NCHUNK = PAIRS_W // CHUNK
GROUP = 4
GW = D * GROUP

N_ROWS = 1000000
WIN = 128
N_FULL = N_ROWS // WIN
REM = N_ROWS - N_FULL * WIN
W_PER = (N_FULL + NW - 1) // NW


def _iota16():
    return lax.broadcasted_iota(jnp.int32, (L,), 0)


def _sqrt16(w):
    bits = plsc.bitcast(w, jnp.int32)
    r = plsc.bitcast(jnp.int32(0x5F3759DF) - (bits >> 1), jnp.float32)
    hw = 0.5 * w
    r = r * (1.5 - hw * r * r)
    r = r * (1.5 - hw * r * r)
    r = r * (1.5 - hw * r * r)
    return w * r


def _neg_acosh1p(t):
    u = t + _sqrt16(t * (2.0 + t))
    poly = 1.0 - u * (0.5 - u * (1.0 / 3.0 - u * (0.25 - u * 0.2)))
    return -(u * poly)


def _transpose_consts():
    iota = _iota16()
    cv = tuple(c0 + iota for c0 in range(0, D, L))
    rot = tuple((iota + jd) & (L - 1) for jd in range(L))
    sa = tuple(r * D + iota for r in rot)
    return cv, rot, sa


def _transpose_block(tc_buf, xr_buf, cv, rot, sa):
    def rblk(r0b, carry):
        r0 = r0b * L
        r0s = r0 * D
        for ci, c0 in enumerate(range(0, D, L)):
            for jd in range(L):
                v = plsc.load_gather(tc_buf, [cv[ci], rot[jd] + r0])
                plsc.store_scatter(xr_buf, [sa[jd] + (r0s + c0)], v)
        return carry

    lax.fori_loop(0, WIN // L, rblk, 0)


def _transpose_body(mt, rem, out, tc0, tc1, xr0, xr1, sem_i0, sem_i1,
                    sem_o0, sem_o1):
    wid = lax.axis_index("s") * NC + lax.axis_index("c")
    bufs = ((tc0, xr0, sem_i0, sem_o0), (tc1, xr1, sem_i1, sem_o1))
    cv, rot, sa = _transpose_consts()

    def win_start(w):
        return w * WIN

    def issue_in(w, tc, sem):
        pltpu.async_copy(mt.at[:, pl.ds(win_start(w), WIN)], tc, sem)

    w0 = wid * W_PER
    nwin = jnp.minimum(jnp.maximum(N_FULL - w0, 0), W_PER)

    @pl.when(nwin > 0)
    def _():
        issue_in(w0, tc0, sem_i0)

    @pl.when(nwin > 1)
    def _():
        issue_in(w0 + 1, tc1, sem_i1)

    def body(i, carry):
        for par, (tc, xr, sem_i, sem_o) in enumerate(bufs):
            j = 2 * i + par
            w = w0 + j

            @pl.when(j < nwin)
            def _():
                pltpu.make_async_copy(
                    mt.at[:, pl.ds(win_start(w), WIN)], tc, sem_i).wait()
                @pl.when(j >= 2)
                def _():
                    pltpu.make_async_copy(
                        xr, out.at[pl.ds((w - 2) * WIN * D, WIN * D)],
                        sem_o).wait()
                _transpose_block(tc, xr, cv, rot, sa)
                nj = j + 2

                @pl.when(nj < nwin)
                def _():
                    issue_in(w0 + nj, tc, sem_i)
                pltpu.async_copy(
                    xr, out.at[pl.ds(w * WIN * D, WIN * D)], sem_o)
        return carry

    lax.fori_loop(0, W_PER // 2 + 1, body, 0)

    for par, (tc, xr, sem_i, sem_o) in enumerate(bufs):
        @pl.when(nwin > par)
        def _():
            last = w0 + nwin - 1
            lastp = jnp.where((nwin - 1) % 2 == par, last, last - 1)
            pltpu.make_async_copy(
                xr, out.at[pl.ds(lastp * WIN * D, WIN * D)], sem_o).wait()

    @pl.when(wid == 0)
    def _():
        pltpu.sync_copy(rem, out.at[pl.ds(N_FULL * WIN * D, REM * D)])


def _pair_body_fn(matrix, items_g, items_lo, origin_g, origin_lo, out,
                  idx_v, lo_v, oidx_v, ylo_v, y_rows, x0, x1,
                  ny_v, out_v, sem_y, sem0, sem1):
    wid = lax.axis_index("s") * NC + lax.axis_index("c")
    pltpu.sync_copy(items_g.at[wid], idx_v)
    pltpu.sync_copy(items_lo.at[wid], lo_v)
    pltpu.sync_copy(origin_g.at[wid], oidx_v)
    pltpu.sync_copy(origin_lo.at[wid], ylo_v)
    y_cp = pltpu.async_copy(matrix.at[oidx_v], y_rows, sem_y)
    pltpu.async_copy(matrix.at[idx_v.at[0]], x0, sem0)
    pltpu.async_copy(matrix.at[idx_v.at[1]], x1, sem1)
    iota = _iota16()
    y_cp.wait()

    for g8 in range(ROWS_W // L):
        rows = iota + (g8 * L)
        lo = plsc.load_gather(ylo_v, [rows])
        acc = jnp.zeros((L,), jnp.float32)
        for d in range(D):
            dd = (iota + d) & (D - 1)
            yd = plsc.load_gather(y_rows, [rows, lo + dd])
            acc = acc + yd * yd
        ny_v[pl.ds(g8 * L, L)] = acc

    def chunk_compute(j, x_buf):
        jv = iota * 0 + j
        for g in range(CHUNK // L):
            rows_x = iota + (g * L)
            p = j * CHUNK + (g * L) + iota
            b = (p * 5243) >> 18
            ny = plsc.load_gather(ny_v, [b])
            ylo = plsc.load_gather(ylo_v, [b])
            xlo = plsc.load_gather(lo_v, [jv, rows_x])
            sq = jnp.zeros((L,), jnp.float32)
            nx = jnp.zeros((L,), jnp.float32)
            for d in range(D):
                dd = (iota + d) & (D - 1)
                xd = plsc.load_gather(x_buf, [rows_x, xlo + dd])
                yd = plsc.load_gather(y_rows, [b, ylo + dd])
                df = xd - yd
                sq = sq + df * df
                nx = nx + xd * xd
            denom = jnp.maximum((1.0 - nx) * (1.0 - ny), 1e-7)
            arg = 1.0 + (2.0 * sq) / denom
            arg = jnp.maximum(arg, 1.0 + 1e-7)
            out_v[pl.ds(j * CHUNK + g * L, L)] = _neg_acosh1p(arg - 1.0)

    def pair_body(i, carry):
        for par, (xb, semb) in enumerate(((x0, sem0), (x1, sem1))):
            j = 2 * i + par
            pltpu.make_async_copy(matrix.at[idx_v.at[j]], xb, semb).wait()
            chunk_compute(j, xb)
            nj = j + 2

            @pl.when(nj < NCHUNK)
            def _():
                pltpu.async_copy(matrix.at[idx_v.at[nj]], xb, semb)
        return carry

    lax.fori_loop(0, NCHUNK // 2, pair_body, 0)
    pltpu.sync_copy(out_v, out.at[pl.ds(wid * PAIRS_W, PAIRS_W)])


def kernel(matrix, items, origin_item):
    mt = matrix.T
    rem = matrix[N_FULL * WIN:, :].reshape(REM * D)
    items_g = (items >> 2).reshape(NW, NCHUNK, CHUNK)
    items_lo = ((items & 3) * D).reshape(NW, NCHUNK, CHUNK)
    origin_g = (origin_item >> 2).reshape(NW, ROWS_W)
    origin_lo = ((origin_item & 3) * D).reshape(NW, ROWS_W)
    mesh = plsc.VectorSubcoreMesh(core_axis_name="c", subcore_axis_name="s")
    f_t = pl.kernel(
        _transpose_body,
        out_type=jax.ShapeDtypeStruct((N_ROWS * D,), jnp.float32),
        mesh=mesh,
        scratch_types=[
            pltpu.VMEM((D, WIN), jnp.float32),
            pltpu.VMEM((D, WIN), jnp.float32),
            pltpu.VMEM((WIN * D,), jnp.float32),
            pltpu.VMEM((WIN * D,), jnp.float32),
            pltpu.SemaphoreType.DMA,
            pltpu.SemaphoreType.DMA,
            pltpu.SemaphoreType.DMA,
            pltpu.SemaphoreType.DMA,
        ],
        compiler_params=pltpu.CompilerParams(
            needs_layout_passes=False, use_tc_tiling_on_sc=True),
    )
    scratch = f_t(mt, rem)
    scratch4 = scratch.reshape(N_ROWS // GROUP, GW)
    f = pl.kernel(
        _pair_body_fn,
        out_type=jax.ShapeDtypeStruct((B * HIST,), jnp.float32),
        mesh=mesh,
        scratch_types=[
            pltpu.VMEM((NCHUNK, CHUNK), jnp.int32),
            pltpu.VMEM((NCHUNK, CHUNK), jnp.int32),
            pltpu.VMEM((ROWS_W,), jnp.int32),
            pltpu.VMEM((ROWS_W,), jnp.int32),
            pltpu.VMEM((ROWS_W, GW), jnp.float32),
            pltpu.VMEM((CHUNK, GW), jnp.float32),
            pltpu.VMEM((CHUNK, GW), jnp.float32),
            pltpu.VMEM((ROWS_W,), jnp.float32),
            pltpu.VMEM((PAIRS_W,), jnp.float32),
            pltpu.SemaphoreType.DMA,
            pltpu.SemaphoreType.DMA,
            pltpu.SemaphoreType.DMA,
        ],
        compiler_params=pltpu.CompilerParams(
            needs_layout_passes=False, use_tc_tiling_on_sc=True),
    )
    out = f(scratch4, items_g, items_lo, origin_g, origin_lo)
    return out.reshape(B, HIST)


# speedup vs baseline: 1.9128x; 1.2492x over previous
import jax
import jax.numpy as jnp
from jax import lax
from jax.experimental import pallas as pl
from jax.experimental.pallas import tpu as pltpu
from jax.experimental.pallas import tpu_sc as plsc

D = 32
B = 4096
HIST = 50
NC = 2
NS = 16
L = 16
NW = NC * NS
ROWS_W = B // NW
PAIRS_W = ROWS_W * HIST
CHUNK = 128
NCHUNK = PAIRS_W // CHUNK
GROUP = 4
GW = D * GROUP

N_ROWS = 1000000
WIN = 128
N_FULL = N_ROWS // WIN
REM = N_ROWS - N_FULL * WIN
W_PER = (N_FULL + NW - 1) // NW


def _iota16():
    return lax.broadcasted_iota(jnp.int32, (L,), 0)


def _sqrt16(w):
    bits = plsc.bitcast(w, jnp.int32)
    r = plsc.bitcast(jnp.int32(0x5F3759DF) - (bits >> 1), jnp.float32)
    hw = 0.5 * w
    r = r * (1.5 - hw * r * r)
    r = r * (1.5 - hw * r * r)
    r = r * (1.5 - hw * r * r)
    return w * r


def _neg_acosh1p(t):
    u = t + _sqrt16(t * (2.0 + t))
    poly = 1.0 - u * (0.5 - u * (1.0 / 3.0 - u * (0.25 - u * 0.2)))
    return -(u * poly)


def _transpose_consts():
    iota = _iota16()
    cv = tuple(c0 + iota for c0 in range(0, D, L))
    rot = tuple((iota + jd) & (L - 1) for jd in range(L))
    sa = tuple(r * D + iota for r in rot)
    return cv, rot, sa


def _transpose_block(tc_buf, xr_buf, cv, rot, sa):
    def rblk(r0b, carry):
        r0 = r0b * L
        r0s = r0 * D
        for ci, c0 in enumerate(range(0, D, L)):
            for j0 in range(0, L, 8):
                vs = [(jd, plsc.load_gather(tc_buf, [cv[ci], rot[jd] + r0]))
                      for jd in range(j0, j0 + 8)]
                for jd, v in vs:
                    plsc.store_scatter(xr_buf, [sa[jd] + (r0s + c0)], v)
        return carry

    lax.fori_loop(0, WIN // L, rblk, 0)


def _transpose_body(mt, rem, out, tc0, tc1, xr0, xr1, sem_i0, sem_i1,
                    sem_o0, sem_o1):
    wid = lax.axis_index("s") * NC + lax.axis_index("c")
    bufs = ((tc0, xr0, sem_i0, sem_o0), (tc1, xr1, sem_i1, sem_o1))
    cv, rot, sa = _transpose_consts()

    def win_start(w):
        return w * WIN

    def issue_in(w, tc, sem):
        pltpu.async_copy(mt.at[:, pl.ds(win_start(w), WIN)], tc, sem)

    w0 = wid * W_PER
    nwin = jnp.minimum(jnp.maximum(N_FULL - w0, 0), W_PER)

    @pl.when(nwin > 0)
    def _():
        issue_in(w0, tc0, sem_i0)

    @pl.when(nwin > 1)
    def _():
        issue_in(w0 + 1, tc1, sem_i1)

    def body(i, carry):
        for par, (tc, xr, sem_i, sem_o) in enumerate(bufs):
            j = 2 * i + par
            w = w0 + j

            @pl.when(j < nwin)
            def _():
                pltpu.make_async_copy(
                    mt.at[:, pl.ds(win_start(w), WIN)], tc, sem_i).wait()
                @pl.when(j >= 2)
                def _():
                    pltpu.make_async_copy(
                        xr, out.at[pl.ds((w - 2) * WIN * D, WIN * D)],
                        sem_o).wait()
                _transpose_block(tc, xr, cv, rot, sa)
                nj = j + 2

                @pl.when(nj < nwin)
                def _():
                    issue_in(w0 + nj, tc, sem_i)
                pltpu.async_copy(
                    xr, out.at[pl.ds(w * WIN * D, WIN * D)], sem_o)
        return carry

    lax.fori_loop(0, W_PER // 2 + 1, body, 0)

    for par, (tc, xr, sem_i, sem_o) in enumerate(bufs):
        @pl.when(nwin > par)
        def _():
            last = w0 + nwin - 1
            lastp = jnp.where((nwin - 1) % 2 == par, last, last - 1)
            pltpu.make_async_copy(
                xr, out.at[pl.ds(lastp * WIN * D, WIN * D)], sem_o).wait()

    @pl.when(wid == 0)
    def _():
        pltpu.sync_copy(rem, out.at[pl.ds(N_FULL * WIN * D, REM * D)])


def _pair_body_fn(matrix, items_g, items_lo, origin_g, origin_lo, out,
                  idx_v, lo_v, oidx_v, ylo_v, y_rows, x0, x1,
                  ny_v, out_v, sem_y, sem0, sem1):
    wid = lax.axis_index("s") * NC + lax.axis_index("c")
    pltpu.sync_copy(items_g.at[wid], idx_v)
    pltpu.sync_copy(items_lo.at[wid], lo_v)
    pltpu.sync_copy(origin_g.at[wid], oidx_v)
    pltpu.sync_copy(origin_lo.at[wid], ylo_v)
    y_cp = pltpu.async_copy(matrix.at[oidx_v], y_rows, sem_y)
    pltpu.async_copy(matrix.at[idx_v.at[0]], x0, sem0)
    pltpu.async_copy(matrix.at[idx_v.at[1]], x1, sem1)
    iota = _iota16()
    y_cp.wait()

    for g8 in range(ROWS_W // L):
        rows = iota + (g8 * L)
        lo = plsc.load_gather(ylo_v, [rows])
        acc = jnp.zeros((L,), jnp.float32)
        for d in range(D):
            dd = (iota + d) & (D - 1)
            yd = plsc.load_gather(y_rows, [rows, lo + dd])
            acc = acc + yd * yd
        ny_v[pl.ds(g8 * L, L)] = acc

    def chunk_compute(j, x_buf):
        jv = iota * 0 + j
        for g in range(CHUNK // L):
            rows_x = iota + (g * L)
            p = j * CHUNK + (g * L) + iota
            b = (p * 5243) >> 18
            ny = plsc.load_gather(ny_v, [b])
            ylo = plsc.load_gather(ylo_v, [b])
            xlo = plsc.load_gather(lo_v, [jv, rows_x])
            sq = jnp.zeros((L,), jnp.float32)
            nx = jnp.zeros((L,), jnp.float32)
            for d in range(D):
                dd = (iota + d) & (D - 1)
                xd = plsc.load_gather(x_buf, [rows_x, xlo + dd])
                yd = plsc.load_gather(y_rows, [b, ylo + dd])
                df = xd - yd
                sq = sq + df * df
                nx = nx + xd * xd
            denom = jnp.maximum((1.0 - nx) * (1.0 - ny), 1e-7)
            arg = 1.0 + (2.0 * sq) / denom
            arg = jnp.maximum(arg, 1.0 + 1e-7)
            out_v[pl.ds(j * CHUNK + g * L, L)] = _neg_acosh1p(arg - 1.0)

    def pair_body(i, carry):
        for par, (xb, semb) in enumerate(((x0, sem0), (x1, sem1))):
            j = 2 * i + par
            pltpu.make_async_copy(matrix.at[idx_v.at[j]], xb, semb).wait()
            chunk_compute(j, xb)
            nj = j + 2

            @pl.when(nj < NCHUNK)
            def _():
                pltpu.async_copy(matrix.at[idx_v.at[nj]], xb, semb)
        return carry

    lax.fori_loop(0, NCHUNK // 2, pair_body, 0)
    pltpu.sync_copy(out_v, out.at[pl.ds(wid * PAIRS_W, PAIRS_W)])


def kernel(matrix, items, origin_item):
    mt = matrix.T
    rem = matrix[N_FULL * WIN:, :].reshape(REM * D)
    items_g = (items >> 2).reshape(NW, NCHUNK, CHUNK)
    items_lo = ((items & 3) * D).reshape(NW, NCHUNK, CHUNK)
    origin_g = (origin_item >> 2).reshape(NW, ROWS_W)
    origin_lo = ((origin_item & 3) * D).reshape(NW, ROWS_W)
    mesh = plsc.VectorSubcoreMesh(core_axis_name="c", subcore_axis_name="s")
    f_t = pl.kernel(
        _transpose_body,
        out_type=jax.ShapeDtypeStruct((N_ROWS * D,), jnp.float32),
        mesh=mesh,
        scratch_types=[
            pltpu.VMEM((D, WIN), jnp.float32),
            pltpu.VMEM((D, WIN), jnp.float32),
            pltpu.VMEM((WIN * D,), jnp.float32),
            pltpu.VMEM((WIN * D,), jnp.float32),
            pltpu.SemaphoreType.DMA,
            pltpu.SemaphoreType.DMA,
            pltpu.SemaphoreType.DMA,
            pltpu.SemaphoreType.DMA,
        ],
        compiler_params=pltpu.CompilerParams(
            needs_layout_passes=False, use_tc_tiling_on_sc=True),
    )
    scratch = f_t(mt, rem)
    scratch4 = scratch.reshape(N_ROWS // GROUP, GW)
    f = pl.kernel(
        _pair_body_fn,
        out_type=jax.ShapeDtypeStruct((B * HIST,), jnp.float32),
        mesh=mesh,
        scratch_types=[
            pltpu.VMEM((NCHUNK, CHUNK), jnp.int32),
            pltpu.VMEM((NCHUNK, CHUNK), jnp.int32),
            pltpu.VMEM((ROWS_W,), jnp.int32),
            pltpu.VMEM((ROWS_W,), jnp.int32),
            pltpu.VMEM((ROWS_W, GW), jnp.float32),
            pltpu.VMEM((CHUNK, GW), jnp.float32),
            pltpu.VMEM((CHUNK, GW), jnp.float32),
            pltpu.VMEM((ROWS_W,), jnp.float32),
            pltpu.VMEM((PAIRS_W,), jnp.float32),
            pltpu.SemaphoreType.DMA,
            pltpu.SemaphoreType.DMA,
            pltpu.SemaphoreType.DMA,
        ],
        compiler_params=pltpu.CompilerParams(
            needs_layout_passes=False, use_tc_tiling_on_sc=True),
    )
    out = f(scratch4, items_g, items_lo, origin_g, origin_lo)
    return out.reshape(B, HIST)


# speedup vs baseline: 1.9365x; 1.0124x over previous
import jax
import jax.numpy as jnp
from jax import lax
from jax.experimental import pallas as pl
from jax.experimental.pallas import tpu as pltpu
from jax.experimental.pallas import tpu_sc as plsc

D = 32
B = 4096
HIST = 50
NC = 2
NS = 16
L = 16
NW = NC * NS
ROWS_W = B // NW
PAIRS_W = ROWS_W * HIST
CHUNK = 128
NCHUNK = PAIRS_W // CHUNK
GROUP = 4
GW = D * GROUP

N_ROWS = 1000000
WIN = 128
N_FULL = N_ROWS // WIN
REM = N_ROWS - N_FULL * WIN
W_PER = (N_FULL + NW - 1) // NW


def _iota16():
    return lax.broadcasted_iota(jnp.int32, (L,), 0)


def _sqrt16(w):
    bits = plsc.bitcast(w, jnp.int32)
    r = plsc.bitcast(jnp.int32(0x5F3759DF) - (bits >> 1), jnp.float32)
    hw = 0.5 * w
    r = r * (1.5 - hw * r * r)
    r = r * (1.5 - hw * r * r)
    r = r * (1.5 - hw * r * r)
    return w * r


def _neg_acosh1p(t):
    u = t + _sqrt16(t * (2.0 + t))
    poly = 1.0 - u * (0.5 - u * (1.0 / 3.0 - u * (0.25 - u * 0.2)))
    return -(u * poly)


def _transpose_consts():
    iota = _iota16()
    cv = tuple(c0 + iota for c0 in range(0, D, L))
    rot = tuple((iota + jd) & (L - 1) for jd in range(L))
    sa = tuple(r * D + iota for r in rot)
    return cv, rot, sa


def _transpose_block(tc_buf, xr_buf, cv, rot, sa):
    def rblk(r0b, carry):
        r0 = r0b * L
        r0s = r0 * D
        for ci, c0 in enumerate(range(0, D, L)):
            for j0 in range(0, L, 8):
                vs = [(jd, plsc.load_gather(tc_buf, [cv[ci], rot[jd] + r0]))
                      for jd in range(j0, j0 + 8)]
                for jd, v in vs:
                    plsc.store_scatter(xr_buf, [sa[jd] + (r0s + c0)], v)
        return carry

    lax.fori_loop(0, WIN // L, rblk, 0)


def _transpose_body(mt, rem, out, tc0, tc1, xr0, xr1, sem_i0, sem_i1,
                    sem_o0, sem_o1):
    wid = lax.axis_index("s") * NC + lax.axis_index("c")
    bufs = ((tc0, xr0, sem_i0, sem_o0), (tc1, xr1, sem_i1, sem_o1))
    cv, rot, sa = _transpose_consts()

    def win_start(w):
        return w * WIN

    def issue_in(w, tc, sem):
        pltpu.async_copy(mt.at[:, pl.ds(win_start(w), WIN)], tc, sem)

    w0 = wid * W_PER
    nwin = jnp.minimum(jnp.maximum(N_FULL - w0, 0), W_PER)

    @pl.when(nwin > 0)
    def _():
        issue_in(w0, tc0, sem_i0)

    @pl.when(nwin > 1)
    def _():
        issue_in(w0 + 1, tc1, sem_i1)

    def body(i, carry):
        for par, (tc, xr, sem_i, sem_o) in enumerate(bufs):
            j = 2 * i + par
            w = w0 + j

            @pl.when(j < nwin)
            def _():
                pltpu.make_async_copy(
                    mt.at[:, pl.ds(win_start(w), WIN)], tc, sem_i).wait()
                @pl.when(j >= 2)
                def _():
                    pltpu.make_async_copy(
                        xr, out.at[pl.ds((w - 2) * WIN * D, WIN * D)],
                        sem_o).wait()
                _transpose_block(tc, xr, cv, rot, sa)
                nj = j + 2

                @pl.when(nj < nwin)
                def _():
                    issue_in(w0 + nj, tc, sem_i)
                pltpu.async_copy(
                    xr, out.at[pl.ds(w * WIN * D, WIN * D)], sem_o)
        return carry

    lax.fori_loop(0, W_PER // 2 + 1, body, 0)

    for par, (tc, xr, sem_i, sem_o) in enumerate(bufs):
        @pl.when(nwin > par)
        def _():
            last = w0 + nwin - 1
            lastp = jnp.where((nwin - 1) % 2 == par, last, last - 1)
            pltpu.make_async_copy(
                xr, out.at[pl.ds(lastp * WIN * D, WIN * D)], sem_o).wait()

    @pl.when(wid == 0)
    def _():
        pltpu.sync_copy(rem, out.at[pl.ds(N_FULL * WIN * D, REM * D)])


def _pair_body_fn(matrix, items_g, items_lo, origin_g, origin_lo, out,
                  idx_v, lo_v, oidx_v, ylo_v, y_rows, x0, x1,
                  ny_v, out_v, sem_y, sem0, sem1):
    wid = lax.axis_index("s") * NC + lax.axis_index("c")
    pltpu.sync_copy(items_g.at[wid], idx_v)
    pltpu.sync_copy(items_lo.at[wid], lo_v)
    pltpu.sync_copy(origin_g.at[wid], oidx_v)
    pltpu.sync_copy(origin_lo.at[wid], ylo_v)
    y_cp = pltpu.async_copy(matrix.at[oidx_v], y_rows, sem_y)
    pltpu.async_copy(matrix.at[idx_v.at[0]], x0, sem0)
    pltpu.async_copy(matrix.at[idx_v.at[1]], x1, sem1)
    iota = _iota16()
    y_cp.wait()

    for g8 in range(ROWS_W // L):
        rows = iota + (g8 * L)
        lo = plsc.load_gather(ylo_v, [rows])
        acc0 = jnp.zeros((L,), jnp.float32)
        acc1 = jnp.zeros((L,), jnp.float32)
        for d0 in range(0, D, 4):
            ys = [plsc.load_gather(y_rows, [rows, lo + ((iota + d) & (D - 1))])
                  for d in range(d0, d0 + 4)]
            acc0 = acc0 + ys[0] * ys[0] + ys[2] * ys[2]
            acc1 = acc1 + ys[1] * ys[1] + ys[3] * ys[3]
        ny_v[pl.ds(g8 * L, L)] = acc0 + acc1

    def chunk_compute(j, x_buf):
        jv = iota * 0 + j
        for g in range(CHUNK // L):
            rows_x = iota + (g * L)
            p = j * CHUNK + (g * L) + iota
            b = (p * 5243) >> 18
            ny = plsc.load_gather(ny_v, [b])
            ylo = plsc.load_gather(ylo_v, [b])
            xlo = plsc.load_gather(lo_v, [jv, rows_x])
            sq0 = jnp.zeros((L,), jnp.float32)
            sq1 = jnp.zeros((L,), jnp.float32)
            nx0 = jnp.zeros((L,), jnp.float32)
            nx1 = jnp.zeros((L,), jnp.float32)
            for d0 in range(0, D, 4):
                dds = [(iota + d) & (D - 1) for d in range(d0, d0 + 4)]
                xs = [plsc.load_gather(x_buf, [rows_x, xlo + dd])
                      for dd in dds]
                ys = [plsc.load_gather(y_rows, [b, ylo + dd]) for dd in dds]
                dfs = [x - y for x, y in zip(xs, ys)]
                sq0 = sq0 + dfs[0] * dfs[0] + dfs[2] * dfs[2]
                sq1 = sq1 + dfs[1] * dfs[1] + dfs[3] * dfs[3]
                nx0 = nx0 + xs[0] * xs[0] + xs[2] * xs[2]
                nx1 = nx1 + xs[1] * xs[1] + xs[3] * xs[3]
            sq = sq0 + sq1
            nx = nx0 + nx1
            denom = jnp.maximum((1.0 - nx) * (1.0 - ny), 1e-7)
            arg = 1.0 + (2.0 * sq) / denom
            arg = jnp.maximum(arg, 1.0 + 1e-7)
            out_v[pl.ds(j * CHUNK + g * L, L)] = _neg_acosh1p(arg - 1.0)

    def pair_body(i, carry):
        for par, (xb, semb) in enumerate(((x0, sem0), (x1, sem1))):
            j = 2 * i + par
            pltpu.make_async_copy(matrix.at[idx_v.at[j]], xb, semb).wait()
            chunk_compute(j, xb)
            nj = j + 2

            @pl.when(nj < NCHUNK)
            def _():
                pltpu.async_copy(matrix.at[idx_v.at[nj]], xb, semb)
        return carry

    lax.fori_loop(0, NCHUNK // 2, pair_body, 0)
    pltpu.sync_copy(out_v, out.at[pl.ds(wid * PAIRS_W, PAIRS_W)])


def kernel(matrix, items, origin_item):
    mt = matrix.T
    rem = matrix[N_FULL * WIN:, :].reshape(REM * D)
    items_g = (items >> 2).reshape(NW, NCHUNK, CHUNK)
    items_lo = ((items & 3) * D).reshape(NW, NCHUNK, CHUNK)
    origin_g = (origin_item >> 2).reshape(NW, ROWS_W)
    origin_lo = ((origin_item & 3) * D).reshape(NW, ROWS_W)
    mesh = plsc.VectorSubcoreMesh(core_axis_name="c", subcore_axis_name="s")
    f_t = pl.kernel(
        _transpose_body,
        out_type=jax.ShapeDtypeStruct((N_ROWS * D,), jnp.float32),
        mesh=mesh,
        scratch_types=[
            pltpu.VMEM((D, WIN), jnp.float32),
            pltpu.VMEM((D, WIN), jnp.float32),
            pltpu.VMEM((WIN * D,), jnp.float32),
            pltpu.VMEM((WIN * D,), jnp.float32),
            pltpu.SemaphoreType.DMA,
            pltpu.SemaphoreType.DMA,
            pltpu.SemaphoreType.DMA,
            pltpu.SemaphoreType.DMA,
        ],
        compiler_params=pltpu.CompilerParams(
            needs_layout_passes=False, use_tc_tiling_on_sc=True),
    )
    scratch = f_t(mt, rem)
    scratch4 = scratch.reshape(N_ROWS // GROUP, GW)
    f = pl.kernel(
        _pair_body_fn,
        out_type=jax.ShapeDtypeStruct((B * HIST,), jnp.float32),
        mesh=mesh,
        scratch_types=[
            pltpu.VMEM((NCHUNK, CHUNK), jnp.int32),
            pltpu.VMEM((NCHUNK, CHUNK), jnp.int32),
            pltpu.VMEM((ROWS_W,), jnp.int32),
            pltpu.VMEM((ROWS_W,), jnp.int32),
            pltpu.VMEM((ROWS_W, GW), jnp.float32),
            pltpu.VMEM((CHUNK, GW), jnp.float32),
            pltpu.VMEM((CHUNK, GW), jnp.float32),
            pltpu.VMEM((ROWS_W,), jnp.float32),
            pltpu.VMEM((PAIRS_W,), jnp.float32),
            pltpu.SemaphoreType.DMA,
            pltpu.SemaphoreType.DMA,
            pltpu.SemaphoreType.DMA,
        ],
        compiler_params=pltpu.CompilerParams(
            needs_layout_passes=False, use_tc_tiling_on_sc=True),
    )
    out = f(scratch4, items_g, items_lo, origin_g, origin_lo)
    return out.reshape(B, HIST)


# speedup vs baseline: 2.1524x; 1.1115x over previous
import jax
import jax.numpy as jnp
from jax import lax
from jax.experimental import pallas as pl
from jax.experimental.pallas import tpu as pltpu
from jax.experimental.pallas import tpu_sc as plsc

D = 32
B = 4096
HIST = 50
NC = 2
NS = 16
L = 16
NW = NC * NS
ROWS_W = B // NW
PAIRS_W = ROWS_W * HIST
CHUNK = 128
NCHUNK = PAIRS_W // CHUNK
GROUP = 4
GW = D * GROUP

N_ROWS = 1000000
WIN = 128
N_FULL = N_ROWS // WIN
REM = N_ROWS - N_FULL * WIN
W_PER = (N_FULL + NW - 1) // NW


def _iota16():
    return lax.broadcasted_iota(jnp.int32, (L,), 0)


def _sqrt16(w):
    bits = plsc.bitcast(w, jnp.int32)
    r = plsc.bitcast(jnp.int32(0x5F3759DF) - (bits >> 1), jnp.float32)
    hw = 0.5 * w
    r = r * (1.5 - hw * r * r)
    r = r * (1.5 - hw * r * r)
    r = r * (1.5 - hw * r * r)
    return w * r


def _neg_acosh1p(t):
    u = t + _sqrt16(t * (2.0 + t))
    poly = 1.0 - u * (0.5 - u * (1.0 / 3.0 - u * (0.25 - u * 0.2)))
    return -(u * poly)


def _transpose_consts():
    iota = _iota16()
    cv = tuple(c0 + iota for c0 in range(0, D, L))
    rot = tuple((iota + jd) & (L - 1) for jd in range(L))
    sa = tuple(r * D + iota for r in rot)
    return cv, rot, sa


def _transpose_block(tc_buf, xr_buf, cv, rot, sa):
    def rblk(r0b, carry):
        r0 = r0b * L
        r0s = r0 * D
        for ci, c0 in enumerate(range(0, D, L)):
            for j0 in range(0, L, 8):
                vs = [(jd, plsc.load_gather(tc_buf, [cv[ci], rot[jd] + r0]))
                      for jd in range(j0, j0 + 8)]
                for jd, v in vs:
                    plsc.store_scatter(xr_buf, [sa[jd] + (r0s + c0)], v)
        return carry

    lax.fori_loop(0, WIN // L, rblk, 0)


def _transpose_body(mt, rem, out, tc0, tc1, xr0, xr1, sem_i0, sem_i1,
                    sem_o0, sem_o1):
    wid = lax.axis_index("s") * NC + lax.axis_index("c")
    bufs = ((tc0, xr0, sem_i0, sem_o0), (tc1, xr1, sem_i1, sem_o1))
    cv, rot, sa = _transpose_consts()

    def win_start(w):
        return w * WIN

    def issue_in(w, tc, sem):
        pltpu.async_copy(mt.at[:, pl.ds(win_start(w), WIN)], tc, sem)

    w0 = wid * W_PER
    nwin = jnp.minimum(jnp.maximum(N_FULL - w0, 0), W_PER)

    @pl.when(nwin > 0)
    def _():
        issue_in(w0, tc0, sem_i0)

    @pl.when(nwin > 1)
    def _():
        issue_in(w0 + 1, tc1, sem_i1)

    def body(i, carry):
        for par, (tc, xr, sem_i, sem_o) in enumerate(bufs):
            j = 2 * i + par
            w = w0 + j

            @pl.when(j < nwin)
            def _():
                pltpu.make_async_copy(
                    mt.at[:, pl.ds(win_start(w), WIN)], tc, sem_i).wait()
                @pl.when(j >= 2)
                def _():
                    pltpu.make_async_copy(
                        xr, out.at[pl.ds((w - 2) * WIN * D, WIN * D)],
                        sem_o).wait()
                _transpose_block(tc, xr, cv, rot, sa)
                nj = j + 2

                @pl.when(nj < nwin)
                def _():
                    issue_in(w0 + nj, tc, sem_i)
                pltpu.async_copy(
                    xr, out.at[pl.ds(w * WIN * D, WIN * D)], sem_o)
        return carry

    lax.fori_loop(0, W_PER // 2 + 1, body, 0)

    for par, (tc, xr, sem_i, sem_o) in enumerate(bufs):
        @pl.when(nwin > par)
        def _():
            last = w0 + nwin - 1
            lastp = jnp.where((nwin - 1) % 2 == par, last, last - 1)
            pltpu.make_async_copy(
                xr, out.at[pl.ds(lastp * WIN * D, WIN * D)], sem_o).wait()

    @pl.when(wid == 0)
    def _():
        pltpu.sync_copy(rem, out.at[pl.ds(N_FULL * WIN * D, REM * D)])


def _pair_body_fn(matrix, items_r, origin_r, out,
                  idx_v, oidx_v, y_rows, x0, x1,
                  ny_v, out_v, sem_y, sem0, sem1):
    wid = lax.axis_index("s") * NC + lax.axis_index("c")
    pltpu.sync_copy(items_r.at[wid], idx_v)
    pltpu.sync_copy(origin_r.at[wid], oidx_v)
    y_cp = pltpu.async_copy(matrix.at[oidx_v], y_rows, sem_y)
    pltpu.async_copy(matrix.at[idx_v.at[0]], x0, sem0)
    pltpu.async_copy(matrix.at[idx_v.at[1]], x1, sem1)
    iota = _iota16()
    y_cp.wait()

    for g8 in range(ROWS_W // L):
        rows = iota + (g8 * L)
        acc0 = jnp.zeros((L,), jnp.float32)
        acc1 = jnp.zeros((L,), jnp.float32)
        for d0 in range(0, D, 4):
            ys = [plsc.load_gather(y_rows, [rows, (iota + d) & (D - 1)])
                  for d in range(d0, d0 + 4)]
            acc0 = acc0 + ys[0] * ys[0] + ys[2] * ys[2]
            acc1 = acc1 + ys[1] * ys[1] + ys[3] * ys[3]
        ny_v[pl.ds(g8 * L, L)] = acc0 + acc1

    def chunk_compute(j, x_buf):
        for g in range(CHUNK // L):
            rows_x = iota + (g * L)
            p = j * CHUNK + (g * L) + iota
            b = (p * 5243) >> 18
            ny = plsc.load_gather(ny_v, [b])
            sq0 = jnp.zeros((L,), jnp.float32)
            sq1 = jnp.zeros((L,), jnp.float32)
            nx0 = jnp.zeros((L,), jnp.float32)
            nx1 = jnp.zeros((L,), jnp.float32)
            for d0 in range(0, D, 4):
                dds = [(iota + d) & (D - 1) for d in range(d0, d0 + 4)]
                xs = [plsc.load_gather(x_buf, [rows_x, dd]) for dd in dds]
                ys = [plsc.load_gather(y_rows, [b, dd]) for dd in dds]
                dfs = [x - y for x, y in zip(xs, ys)]
                sq0 = sq0 + dfs[0] * dfs[0] + dfs[2] * dfs[2]
                sq1 = sq1 + dfs[1] * dfs[1] + dfs[3] * dfs[3]
                nx0 = nx0 + xs[0] * xs[0] + xs[2] * xs[2]
                nx1 = nx1 + xs[1] * xs[1] + xs[3] * xs[3]
            sq = sq0 + sq1
            nx = nx0 + nx1
            denom = jnp.maximum((1.0 - nx) * (1.0 - ny), 1e-7)
            arg = 1.0 + (2.0 * sq) / denom
            arg = jnp.maximum(arg, 1.0 + 1e-7)
            out_v[pl.ds(j * CHUNK + g * L, L)] = _neg_acosh1p(arg - 1.0)

    def pair_body(i, carry):
        for par, (xb, semb) in enumerate(((x0, sem0), (x1, sem1))):
            j = 2 * i + par
            pltpu.make_async_copy(matrix.at[idx_v.at[j]], xb, semb).wait()
            chunk_compute(j, xb)
            nj = j + 2

            @pl.when(nj < NCHUNK)
            def _():
                pltpu.async_copy(matrix.at[idx_v.at[nj]], xb, semb)
        return carry

    lax.fori_loop(0, NCHUNK // 2, pair_body, 0)
    pltpu.sync_copy(out_v, out.at[pl.ds(wid * PAIRS_W, PAIRS_W)])


def kernel(matrix, items, origin_item):
    mt = matrix.T
    rem = matrix[N_FULL * WIN:, :].reshape(REM * D)
    items_r = items.reshape(NW, NCHUNK, CHUNK)
    origin_r = origin_item.reshape(NW, ROWS_W)
    mesh = plsc.VectorSubcoreMesh(core_axis_name="c", subcore_axis_name="s")
    f_t = pl.kernel(
        _transpose_body,
        out_type=jax.ShapeDtypeStruct((N_ROWS * D,), jnp.float32),
        mesh=mesh,
        scratch_types=[
            pltpu.VMEM((D, WIN), jnp.float32),
            pltpu.VMEM((D, WIN), jnp.float32),
            pltpu.VMEM((WIN * D,), jnp.float32),
            pltpu.VMEM((WIN * D,), jnp.float32),
            pltpu.SemaphoreType.DMA,
            pltpu.SemaphoreType.DMA,
            pltpu.SemaphoreType.DMA,
            pltpu.SemaphoreType.DMA,
        ],
        compiler_params=pltpu.CompilerParams(
            needs_layout_passes=False, use_tc_tiling_on_sc=True),
    )
    scratch = f_t(mt, rem)
    scratch2 = scratch.reshape(N_ROWS, D)
    f = pl.kernel(
        _pair_body_fn,
        out_type=jax.ShapeDtypeStruct((B * HIST,), jnp.float32),
        mesh=mesh,
        scratch_types=[
            pltpu.VMEM((NCHUNK, CHUNK), jnp.int32),
            pltpu.VMEM((ROWS_W,), jnp.int32),
            pltpu.VMEM((ROWS_W, D), jnp.float32),
            pltpu.VMEM((CHUNK, D), jnp.float32),
            pltpu.VMEM((CHUNK, D), jnp.float32),
            pltpu.VMEM((ROWS_W,), jnp.float32),
            pltpu.VMEM((PAIRS_W,), jnp.float32),
            pltpu.SemaphoreType.DMA,
            pltpu.SemaphoreType.DMA,
            pltpu.SemaphoreType.DMA,
        ],
        compiler_params=pltpu.CompilerParams(
            needs_layout_passes=False, use_tc_tiling_on_sc=False),
    )
    out = f(scratch2, items_r, origin_r)
    return out.reshape(B, HIST)
